# baseline (speedup 1.0000x reference)
import functools

import jax
import jax.numpy as jnp
from jax import lax
from jax.experimental import pallas as pl
from jax.experimental.pallas import tpu as pltpu
from jax.experimental.pallas import tpu_sc as plsc


def kernel(x, table):
    B, L, D = x.shape
    NC, NS = 2, 16
    NW = NC * NS
    POS_PER_TILE = L // NW
    CH = 16
    NCHUNK = POS_PER_TILE // CH
    W = CH * D

    xf = x.reshape(B, L * D)
    tf = table.reshape(L * D)

    mesh = plsc.VectorSubcoreMesh(core_axis_name="c", subcore_axis_name="s")

    @functools.partial(
        pl.kernel,
        mesh=mesh,
        out_type=jax.ShapeDtypeStruct((B, L * D), jnp.float32),
        scratch_types=[
            pltpu.VMEM((W,), jnp.float32),
            pltpu.VMEM((B, W), jnp.float32),
        ],
    )
    def sc_add(x_hbm, t_hbm, o_hbm, tbuf, xbuf):
        wid = lax.axis_index("s") * NC + lax.axis_index("c")
        base = wid * (POS_PER_TILE * D)

        def chunk(ci, carry):
            off = base + ci * W
            pltpu.sync_copy(t_hbm.at[pl.ds(off, W)], tbuf)
            for b in range(B):
                pltpu.sync_copy(x_hbm.at[b, pl.ds(off, W)], xbuf.at[b])

            def vec(i, c):
                o = i * 16
                t = tbuf[pl.ds(o, 16)]
                for b in range(B):
                    xbuf[b, pl.ds(o, 16)] = xbuf[b, pl.ds(o, 16)] + t
                return c

            lax.fori_loop(0, W // 16, vec, 0)
            for b in range(B):
                pltpu.sync_copy(xbuf.at[b], o_hbm.at[b, pl.ds(off, W)])
            return carry

        lax.fori_loop(0, NCHUNK, chunk, 0)

    out = sc_add(xf, tf)
    return out.reshape(B, L, D)


# speedup vs baseline: 1.2596x; 1.2596x over previous
import functools

import jax
import jax.numpy as jnp
from jax import lax
from jax.experimental import pallas as pl
from jax.experimental.pallas import tpu as pltpu
from jax.experimental.pallas import tpu_sc as plsc


def kernel(x, table):
    B, L, D = x.shape
    NC, NS = 2, 16
    NW = NC * NS
    POS_PER_TILE = L // NW
    CH = 8
    NCHUNK = POS_PER_TILE // CH
    W = CH * D

    xf = x.reshape(B, L * D)
    tf = table.reshape(L * D)

    mesh = plsc.VectorSubcoreMesh(core_axis_name="c", subcore_axis_name="s")

    @functools.partial(
        pl.kernel,
        mesh=mesh,
        out_type=jax.ShapeDtypeStruct((B, L * D), jnp.float32),
        scratch_types=[
            pltpu.VMEM((2, W), jnp.float32),
            pltpu.VMEM((2, B, W), jnp.float32),
            pltpu.SemaphoreType.DMA,
            pltpu.SemaphoreType.DMA,
            pltpu.SemaphoreType.DMA,
            pltpu.SemaphoreType.DMA,
        ],
    )
    def sc_add(x_hbm, t_hbm, o_hbm, tbuf, xbuf, in0, in1, out0, out1):
        wid = lax.axis_index("s") * NC + lax.axis_index("c")
        base = wid * (POS_PER_TILE * D)
        in_sem = [in0, in1]
        out_sem = [out0, out1]

        def start_in(c, s):
            off = base + c * W
            hs = [pltpu.async_copy(t_hbm.at[pl.ds(off, W)], tbuf.at[s], in_sem[s])]
            for b in range(B):
                hs.append(
                    pltpu.async_copy(
                        x_hbm.at[b, pl.ds(off, W)], xbuf.at[s, b], in_sem[s]
                    )
                )
            return hs

        def start_out(c, s):
            off = base + c * W
            return [
                pltpu.async_copy(xbuf.at[s, b], o_hbm.at[b, pl.ds(off, W)], out_sem[s])
                for b in range(B)
            ]

        def compute(s):
            @plsc.parallel_loop(0, W, 16, unroll=8)
            def _(o):
                t = tbuf[s, pl.ds(o, 16)]
                for b in range(B):
                    xbuf[s, b, pl.ds(o, 16)] = xbuf[s, b, pl.ds(o, 16)] + t

        pend_in = [None, None]
        pend_out = [None, None]
        pend_in[0] = start_in(0, 0)
        for c in range(NCHUNK):
            s = c % 2
            ns = 1 - s
            if c + 1 < NCHUNK:
                if pend_out[ns] is not None:
                    for h in pend_out[ns]:
                        h.wait()
                    pend_out[ns] = None
                pend_in[ns] = start_in(c + 1, ns)
            for h in pend_in[s]:
                h.wait()
            compute(s)
            pend_out[s] = start_out(c, s)
        for s in range(2):
            if pend_out[s] is not None:
                for h in pend_out[s]:
                    h.wait()

    return sc_add(xf, tf).reshape(B, L, D)


# speedup vs baseline: 3.6995x; 2.9370x over previous
import functools

import jax
import jax.numpy as jnp
from jax import lax
from jax.experimental import pallas as pl
from jax.experimental.pallas import tpu as pltpu
from jax.experimental.pallas import tpu_sc as plsc


def kernel(x, table):
    B, L, D = x.shape
    NC, NS = 2, 16
    NW = NC * NS
    POS_PER_TILE = L // NW
    CH = 8
    NCHUNK = POS_PER_TILE // CH
    W = CH * D

    x2 = x.reshape(B * L, D)
    mesh = plsc.VectorSubcoreMesh(core_axis_name="c", subcore_axis_name="s")

    @functools.partial(
        pl.kernel,
        mesh=mesh,
        out_type=jax.ShapeDtypeStruct((B * L, D), jnp.float32),
        scratch_types=[
            pltpu.VMEM((2, CH, D), jnp.float32),
            pltpu.VMEM((2, B, CH, D), jnp.float32),
            pltpu.SemaphoreType.DMA,
            pltpu.SemaphoreType.DMA,
            pltpu.SemaphoreType.DMA,
            pltpu.SemaphoreType.DMA,
        ],
        compiler_params=pltpu.CompilerParams(use_tc_tiling_on_sc=True),
    )
    def sc_add(x_hbm, t_hbm, o_hbm, tbuf, xbuf, in0, in1, out0, out1):
        wid = lax.axis_index("s") * NC + lax.axis_index("c")
        base = wid * POS_PER_TILE
        in_sem = [in0, in1]
        out_sem = [out0, out1]

        def start_in(c, s):
            pos = base + c * CH
            hs = [pltpu.async_copy(t_hbm.at[pl.ds(pos, CH), :], tbuf.at[s], in_sem[s])]
            for b in range(B):
                hs.append(
                    pltpu.async_copy(
                        x_hbm.at[pl.ds(b * L + pos, CH), :], xbuf.at[s, b], in_sem[s]
                    )
                )
            return hs

        def start_out(c, s):
            pos = base + c * CH
            return [
                pltpu.async_copy(
                    xbuf.at[s, b], o_hbm.at[pl.ds(b * L + pos, CH), :], out_sem[s]
                )
                for b in range(B)
            ]

        def compute(s):
            @plsc.parallel_loop(0, W, 16, unroll=8)
            def _(i):
                r = i >> 10
                o = pl.multiple_of(i & (D - 1), 16)
                t = tbuf[s, r, pl.ds(o, 16)]
                for b in range(B):
                    xbuf[s, b, r, pl.ds(o, 16)] = xbuf[s, b, r, pl.ds(o, 16)] + t

        pend_in = [None, None]
        pend_out = [None, None]
        pend_in[0] = start_in(0, 0)
        for c in range(NCHUNK):
            s = c % 2
            ns = 1 - s
            if c + 1 < NCHUNK:
                if pend_out[ns] is not None:
                    for h in pend_out[ns]:
                        h.wait()
                    pend_out[ns] = None
                pend_in[ns] = start_in(c + 1, ns)
            for h in pend_in[s]:
                h.wait()
            compute(s)
            pend_out[s] = start_out(c, s)
        for s in range(2):
            if pend_out[s] is not None:
                for h in pend_out[s]:
                    h.wait()

    return sc_add(x2, table).reshape(B, L, D)


# speedup vs baseline: 3.7570x; 1.0155x over previous
import functools

import jax
import jax.numpy as jnp
from jax import lax
from jax.experimental import pallas as pl
from jax.experimental.pallas import tpu as pltpu
from jax.experimental.pallas import tpu_sc as plsc


def kernel(x, table):
    B, L, D = x.shape
    NC, NS = 2, 16
    NW = NC * NS
    POS_PER_TILE = L // NW
    CH = 8
    NCHUNK = POS_PER_TILE // CH
    W = CH * D

    mesh = plsc.VectorSubcoreMesh(core_axis_name="c", subcore_axis_name="s")

    @functools.partial(
        pl.kernel,
        mesh=mesh,
        out_type=jax.ShapeDtypeStruct((B, L, D), jnp.float32),
        scratch_types=[
            pltpu.VMEM((2, CH, D), jnp.float32),
            pltpu.VMEM((2, B, CH, D), jnp.float32),
            pltpu.SemaphoreType.DMA,
            pltpu.SemaphoreType.DMA,
            pltpu.SemaphoreType.DMA,
            pltpu.SemaphoreType.DMA,
        ],
        compiler_params=pltpu.CompilerParams(use_tc_tiling_on_sc=True),
    )
    def sc_add(x_hbm, t_hbm, o_hbm, tbuf, xbuf, in0, in1, out0, out1):
        wid = lax.axis_index("s") * NC + lax.axis_index("c")
        base = wid * POS_PER_TILE
        in_sem = [in0, in1]
        out_sem = [out0, out1]

        def start_in(c, s):
            pos = base + c * CH
            return [
                pltpu.async_copy(t_hbm.at[pl.ds(pos, CH), :], tbuf.at[s], in_sem[s]),
                pltpu.async_copy(
                    x_hbm.at[:, pl.ds(pos, CH), :], xbuf.at[s], in_sem[s]
                ),
            ]

        def start_out(c, s):
            pos = base + c * CH
            return [
                pltpu.async_copy(
                    xbuf.at[s], o_hbm.at[:, pl.ds(pos, CH), :], out_sem[s]
                )
            ]

        def compute(s):
            @plsc.parallel_loop(0, W, 16, unroll=8)
            def _(i):
                r = i >> 10
                o = pl.multiple_of(i & (D - 1), 16)
                t = tbuf[s, r, pl.ds(o, 16)]
                for b in range(B):
                    xbuf[s, b, r, pl.ds(o, 16)] = xbuf[s, b, r, pl.ds(o, 16)] + t

        pend_in = [None, None]
        pend_out = [None, None]
        pend_in[0] = start_in(0, 0)
        for c in range(NCHUNK):
            s = c % 2
            ns = 1 - s
            if c + 1 < NCHUNK:
                if pend_out[ns] is not None:
                    for h in pend_out[ns]:
                        h.wait()
                    pend_out[ns] = None
                pend_in[ns] = start_in(c + 1, ns)
            for h in pend_in[s]:
                h.wait()
            compute(s)
            pend_out[s] = start_out(c, s)
        for s in range(2):
            if pend_out[s] is not None:
                for h in pend_out[s]:
                    h.wait()

    return sc_add(x, table)


# speedup vs baseline: 3.8139x; 1.0152x over previous
import functools

import jax
import jax.numpy as jnp
from jax import lax
from jax.experimental import pallas as pl
from jax.experimental.pallas import tpu as pltpu
from jax.experimental.pallas import tpu_sc as plsc


def kernel(x, table):
    B, L, D = x.shape
    NC, NS = 2, 16
    NW = NC * NS
    POS_PER_TILE = L // NW
    CH = 8
    NCHUNK = POS_PER_TILE // CH
    W = CH * D

    mesh = plsc.VectorSubcoreMesh(core_axis_name="c", subcore_axis_name="s")

    @functools.partial(
        pl.kernel,
        mesh=mesh,
        out_type=jax.ShapeDtypeStruct((B, L, D), jnp.float32),
        scratch_types=[
            pltpu.VMEM((3, CH, D), jnp.float32),
            pltpu.VMEM((3, B, CH, D), jnp.float32),
            pltpu.SemaphoreType.DMA,
            pltpu.SemaphoreType.DMA,
            pltpu.SemaphoreType.DMA,
            pltpu.SemaphoreType.DMA,
            pltpu.SemaphoreType.DMA,
            pltpu.SemaphoreType.DMA,
        ],
        compiler_params=pltpu.CompilerParams(use_tc_tiling_on_sc=True),
    )
    def sc_add(x_hbm, t_hbm, o_hbm, tbuf, xbuf, in0, in1, in2, out0, out1, out2):
        wid = lax.axis_index("s") * NC + lax.axis_index("c")
        base = wid * POS_PER_TILE
        in_sem = [in0, in1, in2]
        out_sem = [out0, out1, out2]

        def start_in(c, s):
            pos = base + c * CH
            return [
                pltpu.async_copy(t_hbm.at[pl.ds(pos, CH), :], tbuf.at[s], in_sem[s]),
                pltpu.async_copy(
                    x_hbm.at[:, pl.ds(pos, CH), :], xbuf.at[s], in_sem[s]
                ),
            ]

        def start_out(c, s):
            pos = base + c * CH
            return [
                pltpu.async_copy(
                    xbuf.at[s], o_hbm.at[:, pl.ds(pos, CH), :], out_sem[s]
                )
            ]

        def compute(s):
            @plsc.parallel_loop(0, W, 16, unroll=8)
            def _(i):
                r = i >> 10
                o = pl.multiple_of(i & (D - 1), 16)
                t = tbuf[s, r, pl.ds(o, 16)]
                for b in range(B):
                    xbuf[s, b, r, pl.ds(o, 16)] = xbuf[s, b, r, pl.ds(o, 16)] + t

        pend_in = [None, None, None]
        pend_out = [None, None, None]
        pend_in[0] = start_in(0, 0)
        pend_in[1] = start_in(1, 1)
        for c in range(NCHUNK):
            s = c % 3
            nc = c + 2
            if nc < NCHUNK:
                ns = nc % 3
                if pend_out[ns] is not None:
                    for h in pend_out[ns]:
                        h.wait()
                    pend_out[ns] = None
                pend_in[ns] = start_in(nc, ns)
            for h in pend_in[s]:
                h.wait()
            compute(s)
            pend_out[s] = start_out(c, s)
        for s in range(3):
            if pend_out[s] is not None:
                for h in pend_out[s]:
                    h.wait()

    return sc_add(x, table)
